# baseline (speedup 1.0000x reference)
import functools

import jax
import jax.numpy as jnp
from jax import lax
from jax.experimental import pallas as pl
from jax.experimental.pallas import tpu as pltpu
from jax.experimental.pallas import tpu_sc as plsc

VOCAB = 1000000
DIM = 64
BATCH = 16384
N_NEG = 20
NPAD = 32

NC, NS = 2, 16
NW = NC * NS

BC_PER_W = BATCH // NW
NEG_PER_W = BATCH * N_NEG // NW

CHUNK = 160
BG_PER_CHUNK = CHUNK // N_NEG
NCH = NEG_PER_W // CHUNK
NBUF = 4

C_CHUNKS = (160, 160, 160, 32)
C_OFFS = (0, 160, 320, 480)


def _dot(rows_v, r, row, t_rows, gb):
    acc = None
    for q in range(4):
        nv = rows_v[r, row, pl.ds(q * 16, 16)]
        tv = t_rows[gb, pl.ds(q * 16, 16)]
        acc = nv * tv if acc is None else acc + nv * tv
    return jnp.sum(acc)


def _sc_target_emb(target, target_table):
    mesh = plsc.VectorSubcoreMesh(core_axis_name="c", subcore_axis_name="s")

    @functools.partial(
        pl.kernel,
        out_type=jax.ShapeDtypeStruct((BATCH, DIM), jnp.float32),
        mesh=mesh,
        compiler_params=pltpu.CompilerParams(use_tc_tiling_on_sc=False,
                                             needs_layout_passes=False),
        scratch_types=[
            pltpu.VMEM((BC_PER_W,), jnp.int32),
            pltpu.VMEM((BC_PER_W, DIM), jnp.float32),
            pltpu.SemaphoreType.DMA,
        ],
    )
    def ka(tgt_hbm, ttab_hbm, temb_out, ti_v, t_rows, tg):
        wid = lax.axis_index("s") * NC + lax.axis_index("c")
        base_tc = wid * BC_PER_W
        pltpu.sync_copy(tgt_hbm.at[pl.ds(base_tc, BC_PER_W)], ti_v)
        pltpu.async_copy(ttab_hbm.at[ti_v], t_rows, tg).wait()
        pltpu.sync_copy(t_rows, temb_out.at[pl.ds(base_tc, BC_PER_W)])

    return ka(target, target_table)


def _sc_scores(t_emb, context, neg_flat, context_table):
    mesh = plsc.VectorSubcoreMesh(core_axis_name="c", subcore_axis_name="s")

    @functools.partial(
        pl.kernel,
        out_type=(
            jax.ShapeDtypeStruct((BATCH,), jnp.float32),
            jax.ShapeDtypeStruct((BATCH, NPAD), jnp.float32),
        ),
        mesh=mesh,
        compiler_params=pltpu.CompilerParams(use_tc_tiling_on_sc=False,
                                             needs_layout_passes=False),
        scratch_types=[
            pltpu.VMEM((BC_PER_W,), jnp.int32),
            pltpu.VMEM((NEG_PER_W,), jnp.int32),
            pltpu.VMEM((NBUF, CHUNK, DIM), jnp.float32),
            pltpu.VMEM((BC_PER_W, DIM), jnp.float32),
            pltpu.VMEM((BC_PER_W,), jnp.float32),
            pltpu.VMEM((BC_PER_W, NPAD), jnp.float32),
            pltpu.SemaphoreType.DMA,
            pltpu.SemaphoreType.DMA,
            pltpu.SemaphoreType.DMA,
            pltpu.SemaphoreType.DMA,
        ],
    )
    def k(temb_hbm, ctx_hbm, neg_hbm, ctab_hbm, pos_out, neg_out,
          tci_v, negidx_v, rows_v, t_rows, pos_v, negs_v, g0, g1, g2, g3):
        g = (g0, g1, g2, g3)
        lane = lax.iota(jnp.int32, 16)
        wid = lax.axis_index("s") * NC + lax.axis_index("c")
        base_tc = wid * BC_PER_W
        base_n = wid * NEG_PER_W

        pltpu.sync_copy(ctx_hbm.at[pl.ds(base_tc, BC_PER_W)], tci_v)
        pltpu.sync_copy(neg_hbm.at[pl.ds(base_n, NEG_PER_W)], negidx_v)

        ch = []
        for r in range(NBUF):
            ch.append(pltpu.async_copy(
                ctab_hbm.at[tci_v.at[pl.ds(C_OFFS[r], C_CHUNKS[r])]],
                rows_v.at[r, pl.ds(0, C_CHUNKS[r])], g[r]))
        pltpu.sync_copy(temb_hbm.at[pl.ds(base_tc, BC_PER_W)], t_rows)

        for r in range(NBUF):
            ch[r].wait()
            coff = C_OFFS[r]

            def pos_body(pg, _, r=r, coff=coff):
                pvec = jnp.zeros((16,), jnp.float32)
                for jj in range(16):
                    row = pg * 16 + jj
                    s = _dot(rows_v, r, row, t_rows, coff + row)
                    pvec = jnp.where(lane == jj, s, pvec)
                pos_v[pl.ds(coff + pg * 16, 16)] = pvec
                return 0

            lax.fori_loop(0, C_CHUNKS[r] // 16, pos_body, 0)
            pltpu.async_copy(
                ctab_hbm.at[negidx_v.at[pl.ds(r * CHUNK, CHUNK)]],
                rows_v.at[r], g[r])

        def neg_iter(i, _):
            for r in range(NBUF):
                kk = i * NBUF + r
                pltpu.make_async_copy(
                    ctab_hbm.at[pl.ds(0, CHUNK)], rows_v.at[r], g[r]).wait()

                def neg_body(g8, _, r=r):
                    gb = kk * BG_PER_CHUNK + g8
                    nv0 = jnp.zeros((16,), jnp.float32)
                    nv1 = jnp.zeros((16,), jnp.float32)
                    for n in range(N_NEG):
                        s = _dot(rows_v, r, g8 * N_NEG + n, t_rows, gb)
                        if n < 16:
                            nv0 = jnp.where(lane == n, s, nv0)
                        else:
                            nv1 = jnp.where(lane == n - 16, s, nv1)
                    negs_v[gb, pl.ds(0, 16)] = nv0
                    negs_v[gb, pl.ds(16, 16)] = nv1
                    return 0

                lax.fori_loop(0, BG_PER_CHUNK, neg_body, 0)

                @pl.when(kk + NBUF < NCH)
                def _():
                    pltpu.async_copy(
                        ctab_hbm.at[negidx_v.at[pl.ds((kk + NBUF) * CHUNK,
                                                      CHUNK)]],
                        rows_v.at[r], g[r])
            return 0

        lax.fori_loop(0, NCH // NBUF, neg_iter, 0)
        pltpu.sync_copy(pos_v, pos_out.at[pl.ds(base_tc, BC_PER_W)])
        pltpu.sync_copy(negs_v, neg_out.at[pl.ds(base_tc, BC_PER_W)])

    return k(t_emb, context, neg_flat, context_table)


PROWS = BATCH // 128
NROWS = BATCH * NPAD // 128


def _tc_loss_body(p_ref, n_ref, out_ref):
    p = p_ref[...]
    val = jnp.sum(jnp.log(jax.nn.sigmoid(p) + 1e-10))
    x = n_ref[...]
    c_io = lax.broadcasted_iota(jnp.int32, (NROWS, 128), 1)
    valid = (c_io % NPAD) < N_NEG
    xs = jnp.where(valid, x, 0.0)
    nl = jnp.log(jax.nn.sigmoid(-xs) + 1e-10)
    val += jnp.sum(jnp.where(valid, nl, 0.0))
    out_ref[...] = jnp.full((1, 1), -1.0 / BATCH, jnp.float32) * val


def _tc_loss(pos2, neg2):
    return pl.pallas_call(
        _tc_loss_body,
        out_shape=jax.ShapeDtypeStruct((1, 1), jnp.float32),
    )(pos2, neg2)


def kernel(target, context, negatives, target_table, context_table):
    target = target.astype(jnp.int32)
    context = context.astype(jnp.int32)
    neg_flat = negatives.astype(jnp.int32).reshape(-1)
    t_emb = _sc_target_emb(target, target_table)
    pos, neg = _sc_scores(t_emb, context, neg_flat, context_table)
    loss = _tc_loss(pos.reshape(PROWS, 128), neg.reshape(NROWS, 128))
    return loss[0, 0]


# speedup vs baseline: 1.0118x; 1.0118x over previous
import functools

import jax
import jax.numpy as jnp
from jax import lax
from jax.experimental import pallas as pl
from jax.experimental.pallas import tpu as pltpu
from jax.experimental.pallas import tpu_sc as plsc

VOCAB = 1000000
DIM = 64
BATCH = 16384
N_NEG = 20
NPAD = 32

NC, NS = 2, 16
NW = NC * NS

BC_PER_W = BATCH // NW
NEG_PER_W = BATCH * N_NEG // NW

CHUNK = 160
BG_PER_CHUNK = CHUNK // N_NEG
NCH = NEG_PER_W // CHUNK
NBUF = 4

C_CHUNKS = (160, 160, 160, 32)
C_OFFS = (0, 160, 320, 480)


def _dot(rows_v, r, row, t_rows, gb):
    acc = None
    for q in range(4):
        nv = rows_v[r, row, pl.ds(q * 16, 16)]
        tv = t_rows[gb, pl.ds(q * 16, 16)]
        acc = nv * tv if acc is None else acc + nv * tv
    return jnp.sum(acc)


def _sc_scores(target, context, neg_flat, target_table, context_table):
    mesh = plsc.VectorSubcoreMesh(core_axis_name="c", subcore_axis_name="s")

    @functools.partial(
        pl.kernel,
        out_type=(
            jax.ShapeDtypeStruct((BATCH,), jnp.float32),
            jax.ShapeDtypeStruct((BATCH, NPAD), jnp.float32),
        ),
        mesh=mesh,
        compiler_params=pltpu.CompilerParams(use_tc_tiling_on_sc=False,
                                             needs_layout_passes=False),
        scratch_types=[
            pltpu.VMEM((2 * BC_PER_W,), jnp.int32),
            pltpu.VMEM((NEG_PER_W,), jnp.int32),
            pltpu.VMEM((NBUF, CHUNK, DIM), jnp.float32),
            pltpu.VMEM((BC_PER_W, DIM), jnp.float32),
            pltpu.VMEM((BC_PER_W,), jnp.float32),
            pltpu.VMEM((BC_PER_W, NPAD), jnp.float32),
            pltpu.SemaphoreType.DMA,
            pltpu.SemaphoreType.DMA,
            pltpu.SemaphoreType.DMA,
            pltpu.SemaphoreType.DMA,
            pltpu.SemaphoreType.DMA,
        ],
    )
    def k(tgt_hbm, ctx_hbm, neg_hbm, ttab_hbm, ctab_hbm, pos_out, neg_out,
          tci_v, negidx_v, rows_v, t_rows, pos_v, negs_v, tg, g0, g1, g2, g3):
        g = (g0, g1, g2, g3)
        lane = lax.iota(jnp.int32, 16)
        wid = lax.axis_index("s") * NC + lax.axis_index("c")
        base_tc = wid * BC_PER_W
        base_n = wid * NEG_PER_W

        pltpu.sync_copy(tgt_hbm.at[pl.ds(base_tc, BC_PER_W)],
                        tci_v.at[pl.ds(0, BC_PER_W)])
        pltpu.sync_copy(ctx_hbm.at[pl.ds(base_tc, BC_PER_W)],
                        tci_v.at[pl.ds(BC_PER_W, BC_PER_W)])
        pltpu.sync_copy(neg_hbm.at[pl.ds(base_n, NEG_PER_W)], negidx_v)

        th = pltpu.async_copy(
            ttab_hbm.at[tci_v.at[pl.ds(0, BC_PER_W)]], t_rows, tg)

        ch = []
        for r in range(NBUF):
            ch.append(pltpu.async_copy(
                ctab_hbm.at[tci_v.at[pl.ds(BC_PER_W + C_OFFS[r], C_CHUNKS[r])]],
                rows_v.at[r, pl.ds(0, C_CHUNKS[r])], g[r]))
        th.wait()

        for r in range(NBUF):
            ch[r].wait()
            coff = C_OFFS[r]

            def pos_body(pg, _, r=r, coff=coff):
                pvec = jnp.zeros((16,), jnp.float32)
                for jj in range(16):
                    row = pg * 16 + jj
                    s = _dot(rows_v, r, row, t_rows, coff + row)
                    pvec = jnp.where(lane == jj, s, pvec)
                pos_v[pl.ds(coff + pg * 16, 16)] = pvec
                return 0

            lax.fori_loop(0, C_CHUNKS[r] // 16, pos_body, 0)
            pltpu.async_copy(
                ctab_hbm.at[negidx_v.at[pl.ds(r * CHUNK, CHUNK)]],
                rows_v.at[r], g[r])

        def neg_iter(i, _):
            for r in range(NBUF):
                kk = i * NBUF + r
                pltpu.make_async_copy(
                    ctab_hbm.at[pl.ds(0, CHUNK)], rows_v.at[r], g[r]).wait()

                def neg_body(g8, _, r=r):
                    gb = kk * BG_PER_CHUNK + g8
                    nv0 = jnp.zeros((16,), jnp.float32)
                    nv1 = jnp.zeros((16,), jnp.float32)
                    for n in range(N_NEG):
                        s = _dot(rows_v, r, g8 * N_NEG + n, t_rows, gb)
                        if n < 16:
                            nv0 = jnp.where(lane == n, s, nv0)
                        else:
                            nv1 = jnp.where(lane == n - 16, s, nv1)
                    negs_v[gb, pl.ds(0, 16)] = nv0
                    negs_v[gb, pl.ds(16, 16)] = nv1
                    return 0

                lax.fori_loop(0, BG_PER_CHUNK, neg_body, 0)

                @pl.when(kk + NBUF < NCH)
                def _():
                    pltpu.async_copy(
                        ctab_hbm.at[negidx_v.at[pl.ds((kk + NBUF) * CHUNK,
                                                      CHUNK)]],
                        rows_v.at[r], g[r])
            return 0

        lax.fori_loop(0, NCH // NBUF, neg_iter, 0)
        pltpu.sync_copy(pos_v, pos_out.at[pl.ds(base_tc, BC_PER_W)])
        pltpu.sync_copy(negs_v, neg_out.at[pl.ds(base_tc, BC_PER_W)])

    return k(target, context, neg_flat, target_table, context_table)


PROWS = BATCH // 128
NROWS = BATCH * NPAD // 128


def _tc_loss_body(p_ref, n_ref, out_ref):
    p = p_ref[...]
    val = jnp.sum(jnp.log(jax.nn.sigmoid(p) + 1e-10))
    x = n_ref[...]
    c_io = lax.broadcasted_iota(jnp.int32, (NROWS, 128), 1)
    valid = (c_io % NPAD) < N_NEG
    xs = jnp.where(valid, x, 0.0)
    nl = jnp.log(jax.nn.sigmoid(-xs) + 1e-10)
    val += jnp.sum(jnp.where(valid, nl, 0.0))
    out_ref[...] = jnp.full((1, 1), -1.0 / BATCH, jnp.float32) * val


def _tc_loss(pos2, neg2):
    return pl.pallas_call(
        _tc_loss_body,
        out_shape=jax.ShapeDtypeStruct((1, 1), jnp.float32),
    )(pos2, neg2)


def kernel(target, context, negatives, target_table, context_table):
    target = target.astype(jnp.int32)
    context = context.astype(jnp.int32)
    neg_flat = negatives.astype(jnp.int32).reshape(-1)
    pos, neg = _sc_scores(target, context, neg_flat,
                          target_table, context_table)
    loss = _tc_loss(pos.reshape(PROWS, 128), neg.reshape(NROWS, 128))
    return loss[0, 0]
